# baseline (speedup 1.0000x reference)
import functools

import jax
import jax.numpy as jnp
from jax import lax
from jax.experimental import pallas as pl
from jax.experimental.pallas import tpu as pltpu, tpu_sc as plsc

_NUM_TOKENS = 16384
_HIDDEN = 4096
_N_EXPERTS = 64
_TOP_K = 8
_N_GROUP = 8
_GROUP_SIZE = 8
_TOPK_GROUP = 4
_SCALE = 2.5

_BT = 1024
_NEG = -1e30

_NW = 32
_CHUNK = _NUM_TOKENS // _NW
_L = 16
_NBATCH = _CHUNK // _L


def _score_kernel(h_ref, wt_ref, b_ref, o_ref):
    logits_t = jax.lax.dot_general(
        wt_ref[...], h_ref[...],
        dimension_numbers=(((1,), (1,)), ((), ())),
        preferred_element_type=jnp.float32,
    )
    o_ref[...] = jax.nn.sigmoid(logits_t) + b_ref[...]


def _tc_scores(hidden_states, wt, b):
    n_tokens = hidden_states.shape[0]
    grid = (n_tokens // _BT,)
    return pl.pallas_call(
        _score_kernel,
        grid=grid,
        in_specs=[
            pl.BlockSpec((_BT, _HIDDEN), lambda i: (i, 0)),
            pl.BlockSpec((_N_EXPERTS, _HIDDEN), lambda i: (0, 0)),
            pl.BlockSpec((_N_EXPERTS, 1), lambda i: (0, 0)),
        ],
        out_specs=pl.BlockSpec((_N_EXPERTS, _BT), lambda i: (0, i)),
        out_shape=jax.ShapeDtypeStruct((_N_EXPERTS, n_tokens), jnp.float32),
    )(hidden_states, wt, b)


def _sc_topk_body(scores_hbm, out_hbm, buf, obuf):
    c = lax.axis_index("c")
    s = lax.axis_index("s")
    wid = s * 2 + c
    base = wid * _CHUNK
    pltpu.sync_copy(scores_hbm.at[:, pl.ds(base, _CHUNK)], buf)

    def batch(j, carry):
        col = j * _L
        v = [buf[e, pl.ds(col, _L)] for e in range(_N_EXPERTS)]

        one = jnp.ones((_L,), dtype=jnp.int32)
        zero = jnp.zeros((_L,), dtype=jnp.int32)
        gsum = []
        for g in range(_N_GROUP):
            vg = v[g * _GROUP_SIZE:(g + 1) * _GROUP_SIZE]
            m1 = vg[0]
            for e in range(1, _GROUP_SIZE):
                m1 = jnp.maximum(m1, vg[e])
            found = zero
            m2 = jnp.full((_L,), _NEG, dtype=jnp.float32)
            for e in range(_GROUP_SIZE):
                hit = jnp.where(vg[e] == m1, one - found, zero)
                found = found | hit
                m2 = jnp.maximum(m2, jnp.where(hit == 1, _NEG, vg[e]))
            gsum.append(m1 + m2)

        sel = [zero for _ in range(_N_GROUP)]
        work = list(gsum)
        for _ in range(_TOPK_GROUP):
            gm = work[0]
            for g in range(1, _N_GROUP):
                gm = jnp.maximum(gm, work[g])
            taken = zero
            for g in range(_N_GROUP):
                hit = jnp.where(work[g] == gm, one - taken, zero)
                taken = taken | hit
                sel[g] = sel[g] | hit
                work[g] = jnp.where(hit == 1, _NEG, work[g])

        cand = [jnp.where(sel[e // _GROUP_SIZE] == 1, v[e], 0.0)
                for e in range(_N_EXPERTS)]
        ws = []
        for _ in range(_TOP_K):
            cm = cand[0]
            for e in range(1, _N_EXPERTS):
                cm = jnp.maximum(cm, cand[e])
            ws.append(cm)
            taken = zero
            for e in range(_N_EXPERTS):
                hit = jnp.where(cand[e] == cm, one - taken, zero)
                taken = taken | hit
                cand[e] = jnp.where(hit == 1, _NEG, cand[e])

        denom = ws[0]
        for i in range(1, _TOP_K):
            denom = denom + ws[i]
        inv = _SCALE / (denom + 1e-20)
        for i in range(_TOP_K):
            obuf[i, pl.ds(col, _L)] = ws[i] * inv
        return carry

    lax.fori_loop(0, _NBATCH, batch, 0)
    pltpu.sync_copy(obuf, out_hbm.at[:, pl.ds(base, _CHUNK)])


def _sc_topk(scores_t):
    mesh = plsc.VectorSubcoreMesh(core_axis_name="c", subcore_axis_name="s")
    f = functools.partial(
        pl.kernel,
        mesh=mesh,
        out_type=jax.ShapeDtypeStruct((_TOP_K, _NUM_TOKENS), jnp.float32),
        scratch_types=[
            pltpu.VMEM((_N_EXPERTS, _CHUNK), jnp.float32),
            pltpu.VMEM((_TOP_K, _CHUNK), jnp.float32),
        ],
    )(_sc_topk_body)
    return f(scores_t)


@jax.jit
def kernel(hidden_states, kernel, e_score_correction_bias):
    wt = kernel.T
    b = e_score_correction_bias.reshape(_N_EXPERTS, 1)
    scores_t = _tc_scores(hidden_states, wt, b)
    out_t = _sc_topk(scores_t)
    return out_t.T
